# Initial kernel scaffold; baseline (speedup 1.0000x reference)
import functools

import jax
import jax.numpy as jnp
from jax import lax
from jax.experimental import pallas as pl
from jax.experimental.pallas import tpu as pltpu
from jax.experimental.pallas import tpu_sc as plsc

_NUM_CORES = 2
_NUM_SUBCORES = 16
_NUM_WORKERS = _NUM_CORES * _NUM_SUBCORES


@functools.cache
def _make_gather(n_rows: int, vocab: int, dim: int):
  assert n_rows % _NUM_WORKERS == 0
  rows_per_w = n_rows // _NUM_WORKERS
  chunk = 3200
  while rows_per_w % chunk:
    chunk //= 2
  n_chunks = rows_per_w // chunk

  mesh = plsc.VectorSubcoreMesh(
      core_axis_name="c", subcore_axis_name="s", num_cores=_NUM_CORES
  )

  @functools.partial(
      pl.kernel,
      mesh=mesh,
      out_type=jax.ShapeDtypeStruct((n_rows, dim), jnp.float32),
      scratch_types=[
          pltpu.VMEM((chunk,), jnp.int32),
          pltpu.VMEM((chunk, dim), jnp.float32),
          pltpu.SemaphoreType.DMA,
      ],
  )
  def gather(table_hbm, idx_hbm, out_hbm, idx_v, rows_v, sem):
    wid = lax.axis_index("s") * _NUM_CORES + lax.axis_index("c")
    base = wid * rows_per_w

    def step(i, _):
      off = base + i * chunk
      pltpu.sync_copy(idx_hbm.at[pl.ds(off, chunk)], idx_v)
      pltpu.async_copy(table_hbm.at[idx_v], rows_v, sem).wait()
      pltpu.sync_copy(rows_v, out_hbm.at[pl.ds(off, chunk)])
      return 0

    lax.fori_loop(0, n_chunks, step, 0)

  return gather


def kernel(x, table):
  b, h = x.shape
  vocab, dim = table.shape
  idx = x.reshape(-1).astype(jnp.int32)
  out = _make_gather(b * h, vocab, dim)(table, idx)
  return out.reshape(b, h, dim)


# speedup vs baseline: 1.1106x; 1.1106x over previous
import functools

import jax
import jax.numpy as jnp
from jax import lax
from jax.experimental import pallas as pl
from jax.experimental.pallas import tpu as pltpu
from jax.experimental.pallas import tpu_sc as plsc

_NUM_CORES = 2
_NUM_SUBCORES = 16
_NUM_WORKERS = _NUM_CORES * _NUM_SUBCORES


@functools.cache
def _make_gather(n_rows: int, vocab: int, dim: int):
  assert n_rows % _NUM_WORKERS == 0
  rows_per_w = n_rows // _NUM_WORKERS
  chunk = 3200
  while rows_per_w % chunk:
    chunk //= 2
  n_chunks = rows_per_w // chunk

  mesh = plsc.VectorSubcoreMesh(
      core_axis_name="c", subcore_axis_name="s", num_cores=_NUM_CORES
  )

  @functools.partial(
      pl.kernel,
      mesh=mesh,
      out_type=jax.ShapeDtypeStruct((n_rows, dim), jnp.float32),
      scratch_types=[
          pltpu.VMEM((chunk,), jnp.int32),
          pltpu.VMEM((chunk, dim), jnp.float32),
          pltpu.SemaphoreType.DMA,
      ],
      compiler_params=pltpu.CompilerParams(use_tc_tiling_on_sc=False),
  )
  def gather(table_hbm, idx_hbm, out_hbm, idx_v, rows_v, sem):
    wid = lax.axis_index("s") * _NUM_CORES + lax.axis_index("c")
    base = wid * rows_per_w

    def step(i, _):
      off = base + i * chunk
      pltpu.sync_copy(idx_hbm.at[pl.ds(off, chunk)], idx_v)
      pltpu.async_copy(table_hbm.at[idx_v], rows_v, sem).wait()
      pltpu.sync_copy(rows_v, out_hbm.at[pl.ds(off, chunk)])
      return 0

    lax.fori_loop(0, n_chunks, step, 0)

  return gather


def kernel(x, table):
  b, h = x.shape
  vocab, dim = table.shape
  idx = x.reshape(-1).astype(jnp.int32)
  out = _make_gather(b * h, vocab, dim)(table, idx)
  return out.reshape(b, h, dim)
